# Initial kernel scaffold; baseline (speedup 1.0000x reference)
import jax
import jax.numpy as jnp
from jax.experimental import pallas as pl


def kernel(x, edge_index, W0, b0, W1, b1):
    raise NotImplementedError("write your pallas kernel here")


# speedup vs baseline: 6.4888x; 6.4888x over previous
import jax
import jax.numpy as jnp
from jax import lax
from jax.experimental import pallas as pl
from jax.experimental.pallas import tpu as pltpu
from jax.experimental.pallas import tpu_sc as plsc

N = 10000
D = 256
H = 128
E = 160000
NC = 2
NS = 16
B = 128
NBATCH = 80
EPT = B * NBATCH
EPAD = EPT * NS
NP = 10240
RPT = NP // NS
ZROWS = 128

BN = 400
NB = N // BN

_mesh = plsc.VectorSubcoreMesh(core_axis_name="c", subcore_axis_name="s")


def _deg_body(dstr_hbm, out_hbm, idx_dst, ones_b, acc):
    c = lax.axis_index("c")
    s = lax.axis_index("s")

    @pl.when(c == 0)
    def _():
        pltpu.sync_copy(dstr_hbm.at[s], idx_dst)

        one = jnp.ones((16,), jnp.float32)
        zero = jnp.zeros((16,), jnp.float32)

        def fillz(j, _):
            for k in range(H // 16):
                ones_b[j, pl.ds(k * 16, 16)] = zero
            return 0
        lax.fori_loop(0, ZROWS, fillz, 0)
        for q in range(RPT // ZROWS):
            pltpu.sync_copy(ones_b, acc.at[pl.ds(s * RPT + q * ZROWS, ZROWS)])

        def fill(j, _):
            for k in range(H // 16):
                ones_b[j, pl.ds(k * 16, 16)] = one
            return 0
        lax.fori_loop(0, B, fill, 0)
        plsc.subcore_barrier()

        def batch(j, _):
            pltpu.sync_copy(ones_b, acc.at[idx_dst.at[j]], add=True)
            return 0
        lax.fori_loop(0, NBATCH, batch, 0)
        plsc.subcore_barrier()

        pltpu.sync_copy(acc.at[pl.ds(s * RPT, RPT)],
                        out_hbm.at[pl.ds(s * RPT, RPT)])


_deg_call = pl.kernel(
    _deg_body,
    out_type=jax.ShapeDtypeStruct((NP, H), jnp.float32),
    mesh=_mesh,
    scratch_types=[
        pltpu.VMEM((NBATCH, B), jnp.int32),
        pltpu.VMEM((B, H), jnp.float32),
        pltpu.VMEM_SHARED((NP, H), jnp.float32),
    ],
)


def _scatter_body(y_hbm, srcr_hbm, dstr_hbm, out_hbm,
                  idx_src, idx_dst, rows, acc, sem):
    c = lax.axis_index("c")
    s = lax.axis_index("s")

    pltpu.sync_copy(srcr_hbm.at[s], idx_src)
    pltpu.sync_copy(dstr_hbm.at[s], idx_dst)

    off = c * NP

    def addoff(j, _):
        for k in range(B // 16):
            sl = pl.ds(k * 16, 16)
            idx_src[j, sl] = idx_src[j, sl] + off
        return 0
    lax.fori_loop(0, NBATCH, addoff, 0)

    zero = jnp.zeros((16,), jnp.float32)

    def zrow(j, _):
        for k in range(H // 16):
            rows[j, pl.ds(k * 16, 16)] = zero
        return 0
    lax.fori_loop(0, ZROWS, zrow, 0)
    for q in range(RPT // ZROWS):
        pltpu.sync_copy(rows, acc.at[pl.ds(s * RPT + q * ZROWS, ZROWS)])
    plsc.subcore_barrier()

    def batch(j, _):
        pltpu.async_copy(y_hbm.at[idx_src.at[j]], rows, sem).wait()
        pltpu.sync_copy(rows, acc.at[idx_dst.at[j]], add=True)
        return 0
    lax.fori_loop(0, NBATCH, batch, 0)
    plsc.subcore_barrier()

    pltpu.sync_copy(acc.at[pl.ds(s * RPT, RPT)],
                    out_hbm.at[pl.ds(off + s * RPT, RPT)])


_scatter_call = pl.kernel(
    _scatter_body,
    out_type=jax.ShapeDtypeStruct((NC * NP, H), jnp.float32),
    mesh=_mesh,
    scratch_types=[
        pltpu.VMEM((NBATCH, B), jnp.int32),
        pltpu.VMEM((NBATCH, B), jnp.int32),
        pltpu.VMEM((B, H), jnp.float32),
        pltpu.VMEM_SHARED((NP, H), jnp.float32),
        pltpu.SemaphoreType.DMA,
    ],
)


def _k0_body(x_ref, w_ref, deg_ref, y_ref):
    dinv = lax.rsqrt(deg_ref[:, 0:1] + 1.0)
    y_ref[0] = jnp.dot(x_ref[...], w_ref[...],
                       preferred_element_type=jnp.float32) * dinv


_k0_call = pl.pallas_call(
    _k0_body,
    grid=(NB, NC),
    in_specs=[
        pl.BlockSpec((BN, D), lambda i, c: (i, 0)),
        pl.BlockSpec((D, H), lambda i, c: (0, c)),
        pl.BlockSpec((BN, H), lambda i, c: (i, 0)),
    ],
    out_specs=pl.BlockSpec((1, BN, H), lambda i, c: (c, i, 0)),
    out_shape=jax.ShapeDtypeStruct((NC, NP, H), jnp.float32),
)


def _k1_body(a0_ref, a1_ref, y0_ref, y1_ref, deg_ref, w_ref, b_ref, o_ref):
    dinv = lax.rsqrt(deg_ref[:, 0:1] + 1.0)
    t = jnp.concatenate([a0_ref[0] + y0_ref[0],
                         a1_ref[0] + y1_ref[0]], axis=1)
    h = jnp.maximum(t * dinv + b_ref[...], 0.0)
    o_ref[0] = jnp.dot(h, w_ref[...],
                       preferred_element_type=jnp.float32) * dinv


_k1_call = pl.pallas_call(
    _k1_body,
    grid=(NB, NC),
    in_specs=[
        pl.BlockSpec((1, BN, H), lambda i, c: (0, i, 0)),
        pl.BlockSpec((1, BN, H), lambda i, c: (1, i, 0)),
        pl.BlockSpec((1, BN, H), lambda i, c: (0, i, 0)),
        pl.BlockSpec((1, BN, H), lambda i, c: (1, i, 0)),
        pl.BlockSpec((BN, H), lambda i, c: (i, 0)),
        pl.BlockSpec((D, H), lambda i, c: (0, c)),
        pl.BlockSpec((1, D), lambda i, c: (0, 0)),
    ],
    out_specs=pl.BlockSpec((1, BN, H), lambda i, c: (c, i, 0)),
    out_shape=jax.ShapeDtypeStruct((NC, NP, H), jnp.float32),
)


def _k2_body(a0_ref, a1_ref, y0_ref, y1_ref, deg_ref, b_ref, o_ref):
    dinv = lax.rsqrt(deg_ref[:, 0:1] + 1.0)
    t = jnp.concatenate([a0_ref[0] + y0_ref[0],
                         a1_ref[0] + y1_ref[0]], axis=1)
    o_ref[...] = t * dinv + b_ref[...]


_k2_call = pl.pallas_call(
    _k2_body,
    grid=(NB,),
    in_specs=[
        pl.BlockSpec((1, BN, H), lambda i: (0, i, 0)),
        pl.BlockSpec((1, BN, H), lambda i: (1, i, 0)),
        pl.BlockSpec((1, BN, H), lambda i: (0, i, 0)),
        pl.BlockSpec((1, BN, H), lambda i: (1, i, 0)),
        pl.BlockSpec((BN, H), lambda i: (i, 0)),
        pl.BlockSpec((1, D), lambda i: (0, 0)),
    ],
    out_specs=pl.BlockSpec((BN, D), lambda i: (i, 0)),
    out_shape=jax.ShapeDtypeStruct((N, D), jnp.float32),
)


def kernel(x, edge_index, W0, b0, W1, b1):
    src = edge_index[0]
    dst = edge_index[1]
    pad = EPAD - E
    srcp = jnp.concatenate(
        [src, jnp.zeros((pad,), jnp.int32)]).reshape(NS, NBATCH, B)
    dstp = jnp.concatenate(
        [dst, jnp.full((pad,), N, jnp.int32)]).reshape(NS, NBATCH, B)

    deg16 = _deg_call(dstp)
    y1 = _k0_call(x, W0, deg16)
    agg1 = _scatter_call(y1.reshape(NC * NP, H), srcp, dstp).reshape(NC, NP, H)
    y2 = _k1_call(agg1, agg1, y1, y1, deg16, W1, b0.reshape(1, D))
    agg2 = _scatter_call(y2.reshape(NC * NP, H), srcp, dstp).reshape(NC, NP, H)
    out = _k2_call(agg2, agg2, y2, y2, deg16, b1.reshape(1, D))
    return out


# speedup vs baseline: 7.5874x; 1.1693x over previous
import jax
import jax.numpy as jnp
from jax import lax
from jax.experimental import pallas as pl
from jax.experimental.pallas import tpu as pltpu
from jax.experimental.pallas import tpu_sc as plsc

N = 10000
D = 256
H = 128
E = 160000
NC = 2
NS = 16
B = 128
NBATCH = 80
EPT = B * NBATCH
EPAD = EPT * NS
NP = 10240
RPT = NP // NS
ZROWS = 128

NCHUNK = 2
CB = NBATCH // NCHUNK

BN = 400
NB = N // BN

_mesh = plsc.VectorSubcoreMesh(core_axis_name="c", subcore_axis_name="s")


def _deg_body(dstr_hbm, out_hbm, idx_dst, ones_b, acc):
    c = lax.axis_index("c")
    s = lax.axis_index("s")

    @pl.when(c == 0)
    def _():
        for q in range(NCHUNK):
            pltpu.sync_copy(dstr_hbm.at[s * NCHUNK + q],
                            idx_dst.at[pl.ds(q * CB, CB)])

        one = jnp.ones((16,), jnp.float32)
        zero = jnp.zeros((16,), jnp.float32)

        def fillz(j, _):
            for k in range(H // 16):
                ones_b[j, pl.ds(k * 16, 16)] = zero
            return 0
        lax.fori_loop(0, ZROWS, fillz, 0)
        for q in range(RPT // ZROWS):
            pltpu.sync_copy(ones_b, acc.at[pl.ds(s * RPT + q * ZROWS, ZROWS)])

        def fill(j, _):
            for k in range(H // 16):
                ones_b[j, pl.ds(k * 16, 16)] = one
            return 0
        lax.fori_loop(0, B, fill, 0)
        plsc.subcore_barrier()

        def batch(j, _):
            pltpu.sync_copy(ones_b, acc.at[idx_dst.at[j]], add=True)
            return 0
        lax.fori_loop(0, NBATCH, batch, 0)
        plsc.subcore_barrier()

        pltpu.sync_copy(acc.at[pl.ds(s * RPT, RPT)],
                        out_hbm.at[pl.ds(s * RPT, RPT)])


_deg_call = pl.kernel(
    _deg_body,
    out_type=jax.ShapeDtypeStruct((NP, H), jnp.float32),
    mesh=_mesh,
    scratch_types=[
        pltpu.VMEM((NBATCH, B), jnp.int32),
        pltpu.VMEM((B, H), jnp.float32),
        pltpu.VMEM_SHARED((NP, H), jnp.float32),
    ],
)


def _scatter_body(y_hbm, srcr_hbm, dstr_hbm, out_hbm,
                  idx_src, idx_dst, rows0, rows1, acc, sem0, sem1):
    c = lax.axis_index("c")
    s = lax.axis_index("s")
    off = c * NP
    zero = jnp.zeros((16,), jnp.float32)

    def zrow(j, _):
        for k in range(H // 16):
            rows0[j, pl.ds(k * 16, 16)] = zero
        return 0
    lax.fori_loop(0, ZROWS, zrow, 0)
    for q in range(RPT // ZROWS):
        pltpu.sync_copy(rows0, acc.at[pl.ds(s * RPT + q * ZROWS, ZROWS)])
    plsc.subcore_barrier()

    def chunk(cc, _):
        pltpu.sync_copy(srcr_hbm.at[s * NCHUNK + cc], idx_src)
        pltpu.sync_copy(dstr_hbm.at[s * NCHUNK + cc], idx_dst)

        def addoff(j, _):
            for k in range(B // 16):
                sl = pl.ds(k * 16, 16)
                idx_src[j, sl] = idx_src[j, sl] + off
            return 0
        lax.fori_loop(0, CB, addoff, 0)

        pltpu.async_copy(y_hbm.at[idx_src.at[0]], rows0, sem0)

        def pipe(jj, _):
            j = jj * 2
            pltpu.async_copy(y_hbm.at[idx_src.at[j + 1]], rows1, sem1)
            pltpu.make_async_copy(y_hbm.at[idx_src.at[j]], rows0, sem0).wait()
            pltpu.sync_copy(rows0, acc.at[idx_dst.at[j]], add=True)

            @pl.when(jj < CB // 2 - 1)
            def _():
                pltpu.async_copy(y_hbm.at[idx_src.at[j + 2]], rows0, sem0)
            pltpu.make_async_copy(y_hbm.at[idx_src.at[j + 1]], rows1, sem1).wait()
            pltpu.sync_copy(rows1, acc.at[idx_dst.at[j + 1]], add=True)
            return 0
        lax.fori_loop(0, CB // 2, pipe, 0)
        return 0
    lax.fori_loop(0, NCHUNK, chunk, 0)
    plsc.subcore_barrier()

    pltpu.sync_copy(acc.at[pl.ds(s * RPT, RPT)],
                    out_hbm.at[pl.ds(off + s * RPT, RPT)])


_scatter_call = pl.kernel(
    _scatter_body,
    out_type=jax.ShapeDtypeStruct((NC * NP, H), jnp.float32),
    mesh=_mesh,
    scratch_types=[
        pltpu.VMEM((CB, B), jnp.int32),
        pltpu.VMEM((CB, B), jnp.int32),
        pltpu.VMEM((B, H), jnp.float32),
        pltpu.VMEM((B, H), jnp.float32),
        pltpu.VMEM_SHARED((NP, H), jnp.float32),
        pltpu.SemaphoreType.DMA,
        pltpu.SemaphoreType.DMA,
    ],
)


def _k0_body(x_ref, w_ref, deg_ref, y_ref):
    dinv = lax.rsqrt(deg_ref[:, 0:1] + 1.0)
    y_ref[0] = jnp.dot(x_ref[...], w_ref[...],
                       preferred_element_type=jnp.float32) * dinv


_k0_call = pl.pallas_call(
    _k0_body,
    grid=(NB, NC),
    in_specs=[
        pl.BlockSpec((BN, D), lambda i, c: (i, 0)),
        pl.BlockSpec((D, H), lambda i, c: (0, c)),
        pl.BlockSpec((BN, H), lambda i, c: (i, 0)),
    ],
    out_specs=pl.BlockSpec((1, BN, H), lambda i, c: (c, i, 0)),
    out_shape=jax.ShapeDtypeStruct((NC, NP, H), jnp.float32),
)


def _k1_body(a0_ref, a1_ref, y0_ref, y1_ref, deg_ref, w_ref, b_ref, o_ref):
    dinv = lax.rsqrt(deg_ref[:, 0:1] + 1.0)
    t = jnp.concatenate([a0_ref[0] + y0_ref[0],
                         a1_ref[0] + y1_ref[0]], axis=1)
    h = jnp.maximum(t * dinv + b_ref[...], 0.0)
    o_ref[0] = jnp.dot(h, w_ref[...],
                       preferred_element_type=jnp.float32) * dinv


_k1_call = pl.pallas_call(
    _k1_body,
    grid=(NB, NC),
    in_specs=[
        pl.BlockSpec((1, BN, H), lambda i, c: (0, i, 0)),
        pl.BlockSpec((1, BN, H), lambda i, c: (1, i, 0)),
        pl.BlockSpec((1, BN, H), lambda i, c: (0, i, 0)),
        pl.BlockSpec((1, BN, H), lambda i, c: (1, i, 0)),
        pl.BlockSpec((BN, H), lambda i, c: (i, 0)),
        pl.BlockSpec((D, H), lambda i, c: (0, c)),
        pl.BlockSpec((1, D), lambda i, c: (0, 0)),
    ],
    out_specs=pl.BlockSpec((1, BN, H), lambda i, c: (c, i, 0)),
    out_shape=jax.ShapeDtypeStruct((NC, NP, H), jnp.float32),
)


def _k2_body(a0_ref, a1_ref, y0_ref, y1_ref, deg_ref, b_ref, o_ref):
    dinv = lax.rsqrt(deg_ref[:, 0:1] + 1.0)
    t = jnp.concatenate([a0_ref[0] + y0_ref[0],
                         a1_ref[0] + y1_ref[0]], axis=1)
    o_ref[...] = t * dinv + b_ref[...]


_k2_call = pl.pallas_call(
    _k2_body,
    grid=(NB,),
    in_specs=[
        pl.BlockSpec((1, BN, H), lambda i: (0, i, 0)),
        pl.BlockSpec((1, BN, H), lambda i: (1, i, 0)),
        pl.BlockSpec((1, BN, H), lambda i: (0, i, 0)),
        pl.BlockSpec((1, BN, H), lambda i: (1, i, 0)),
        pl.BlockSpec((BN, H), lambda i: (i, 0)),
        pl.BlockSpec((1, D), lambda i: (0, 0)),
    ],
    out_specs=pl.BlockSpec((BN, D), lambda i: (i, 0)),
    out_shape=jax.ShapeDtypeStruct((N, D), jnp.float32),
)


def kernel(x, edge_index, W0, b0, W1, b1):
    src = edge_index[0]
    dst = edge_index[1]
    pad = EPAD - E
    srcp = jnp.concatenate(
        [src, jnp.zeros((pad,), jnp.int32)]).reshape(NS * NCHUNK, CB, B)
    dstp = jnp.concatenate(
        [dst, jnp.full((pad,), N, jnp.int32)]).reshape(NS * NCHUNK, CB, B)

    deg16 = _deg_call(dstp)
    y1 = _k0_call(x, W0, deg16)
    agg1 = _scatter_call(y1.reshape(NC * NP, H), srcp, dstp).reshape(NC, NP, H)
    y2 = _k1_call(agg1, agg1, y1, y1, deg16, W1, b0.reshape(1, D))
    agg2 = _scatter_call(y2.reshape(NC * NP, H), srcp, dstp).reshape(NC, NP, H)
    out = _k2_call(agg2, agg2, y2, y2, deg16, b1.reshape(1, D))
    return out
